# baseline (speedup 1.0000x reference)
import functools

import jax
import jax.numpy as jnp
from jax import lax
from jax.experimental import pallas as pl
from jax.experimental.pallas import tpu as pltpu
from jax.experimental.pallas import tpu_sc as plsc

B = 4096
F = 26
DW = 16
DD = 16
D_IN = F * DD

_info = plsc.get_sparse_core_info()
NC, NS, L = _info.num_cores, _info.num_subcores, _info.num_lanes
NW = NC * NS
BPW = B // NW
IDX_CHUNK = 128
NCHUNK = BPW * F // IDX_CHUNK


def _sc_gather_body(x_flat, wide_tab, deep_tab, deep_out, wide_out,
                    idx_v, drows_v, wrows_v, wpart_v, sem_d, sem_w):
    wid = lax.axis_index("s") * NC + lax.axis_index("c")
    pltpu.sync_copy(x_flat.at[pl.ds(wid * BPW * F, BPW * F)], idx_v)
    handles = []
    for k in range(NCHUNK):
        sl = pl.ds(k * IDX_CHUNK, IDX_CHUNK)
        handles.append(pltpu.async_copy(deep_tab.at[idx_v.at[sl]],
                                        drows_v.at[sl], sem_d))
        handles.append(pltpu.async_copy(wide_tab.at[idx_v.at[sl]],
                                        wrows_v.at[sl], sem_w))
    for h in handles:
        h.wait()

    def body(i, _):
        base = i * F
        acc = wrows_v[base, :]
        for f in range(1, F):
            acc = acc + wrows_v[base + f, :]
        wpart_v[i, :] = acc
        return 0

    lax.fori_loop(0, BPW, body, 0)

    pltpu.sync_copy(drows_v, deep_out.at[pl.ds(wid * BPW * F, BPW * F)])
    pltpu.sync_copy(wpart_v, wide_out.at[pl.ds(wid * BPW, BPW)])


def _sc_gather(x2, wide_tab, deep_tab):
    mesh = plsc.VectorSubcoreMesh(core_axis_name="c", subcore_axis_name="s")
    fn = pl.kernel(
        _sc_gather_body,
        mesh=mesh,
        compiler_params=pltpu.CompilerParams(use_tc_tiling_on_sc=False),
        out_type=[
            jax.ShapeDtypeStruct((B * F, DD), jnp.float32),
            jax.ShapeDtypeStruct((B, DW), jnp.float32),
        ],
        scratch_types=[
            pltpu.VMEM((BPW * F,), jnp.int32),
            pltpu.VMEM((BPW * F, DD), jnp.float32),
            pltpu.VMEM((BPW * F, DW), jnp.float32),
            pltpu.VMEM((BPW, DW), jnp.float32),
            pltpu.SemaphoreType.DMA,
            pltpu.SemaphoreType.DMA,
        ],
    )
    return fn(x2, wide_tab, deep_tab)


CHUNK_B = 512
NB = B // CHUNK_B


def _mlp_body(deep_ref, wide_ref, gamma_ref, beta_ref,
              W1_ref, b1_ref, W2_ref, b2_ref, W3_ref, b3_ref, W4_ref, b4_ref,
              out_ref, sum_ref, sq_ref):
    ph = pl.program_id(0)
    c = pl.program_id(1)

    @pl.when(jnp.logical_and(ph == 0, c == 0))
    def _init():
        sum_ref[...] = jnp.zeros_like(sum_ref)
        sq_ref[...] = jnp.zeros_like(sq_ref)

    @pl.when(ph == 0)
    def _stats():
        d = deep_ref[...]
        sum_ref[...] += jnp.sum(d, axis=0, keepdims=True)
        sq_ref[...] += jnp.sum(d * d, axis=0, keepdims=True)

    @pl.when(ph == 1)
    def _mlp():
        inv_b = 1.0 / B
        mean = sum_ref[...] * inv_b
        ex2 = sq_ref[...] * inv_b
        var = ex2 - mean * mean
        scale = gamma_ref[...] * lax.rsqrt(var + 1e-5)
        shift = beta_ref[...] - mean * scale
        h = deep_ref[...] * scale + shift
        h = jnp.maximum(jnp.dot(h, W1_ref[...],
                                preferred_element_type=jnp.float32)
                        + b1_ref[...], 0.0)
        h = jnp.maximum(jnp.dot(h, W2_ref[...],
                                preferred_element_type=jnp.float32)
                        + b2_ref[...], 0.0)
        h = jnp.maximum(jnp.dot(h, W3_ref[...],
                                preferred_element_type=jnp.float32)
                        + b3_ref[...], 0.0)
        z = jnp.dot(h, W4_ref[...],
                    preferred_element_type=jnp.float32) + b4_ref[...]
        wide_o = jnp.sum(wide_ref[...], axis=1, keepdims=True)
        out_ref[...] = jax.nn.sigmoid(z + wide_o)


def _tc_mlp(deep_emb, wide_part, gamma, beta, W1, b1, W2, b2, W3, b3, W4, b4):
    full = lambda shape: pl.BlockSpec(shape, lambda p, c: (0,) * len(shape))
    grid_spec = pltpu.PrefetchScalarGridSpec(
        num_scalar_prefetch=0,
        grid=(2, NB),
        in_specs=[
            pl.BlockSpec((CHUNK_B, D_IN), lambda p, c: (c, 0)),
            pl.BlockSpec((CHUNK_B, DW), lambda p, c: (c, 0)),
            full((1, D_IN)), full((1, D_IN)),
            full(W1.shape), full((1, 512)),
            full(W2.shape), full((1, 256)),
            full(W3.shape), full((1, 128)),
            full(W4.shape), full((1, 1)),
        ],
        out_specs=pl.BlockSpec((CHUNK_B, 1), lambda p, c: (c, 0)),
        scratch_shapes=[
            pltpu.VMEM((1, D_IN), jnp.float32),
            pltpu.VMEM((1, D_IN), jnp.float32),
        ],
    )
    return pl.pallas_call(
        _mlp_body,
        grid_spec=grid_spec,
        out_shape=jax.ShapeDtypeStruct((B, 1), jnp.float32),
    )(deep_emb, wide_part, gamma.reshape(1, -1), beta.reshape(1, -1),
      W1, b1.reshape(1, -1), W2, b2.reshape(1, -1),
      W3, b3.reshape(1, -1), W4, b4.reshape(1, -1))


def kernel(x, wide_table, deep_table, gamma, beta,
           W1, b1, W2, b2, W3, b3, W4, b4):
    x_flat = x.reshape(B * F).astype(jnp.int32)
    deep_rows, wide_part = _sc_gather(x_flat, wide_table, deep_table)
    deep_emb = deep_rows.reshape(B, D_IN)
    return _tc_mlp(deep_emb, wide_part, gamma, beta,
                   W1, b1, W2, b2, W3, b3, W4, b4)
